# baseline (speedup 1.0000x reference)
import functools

import jax
import jax.numpy as jnp
from jax import lax
from jax.experimental import pallas as pl
from jax.experimental.pallas import tpu as pltpu, tpu_sc as plsc

N = 4096
D = 64
PERPLEXITY = 30.0
TOL = 1e-05
K = int(PERPLEXITY * 3)
N_CATEGORIES = 8
N_TRIES = 50

G = 4
GN = N // G
TILE_A = 256
TILE_C = 512
PAD = 128
XW = 2 * N
NW = 32
RPW = GN // NW
L = 16
NB = 128
NBLK = N // NB

_MAX_FINITE_BITS = 0x7F7FFFFF
_PADVAL = 1e38


def _tc_a_kernel(goff, x_ref, xt_ref, bl_ref, out_ref):
    pid = pl.program_id(0)
    x = x_ref[...]
    xt = xt_ref[...]

    dot = lax.dot_general(
        x, xt, (((1,), (0,)), ((), ())), preferred_element_type=jnp.float32)
    sq_row = jnp.sum(x * x, axis=1, keepdims=True)
    sq_lane = jnp.sum(xt * xt, axis=0, keepdims=True)
    d2 = jnp.maximum(sq_row + sq_lane - 2.0 * dot, 0.0)

    row_ids = goff + pid * TILE_A + lax.broadcasted_iota(
        jnp.int32, (TILE_A, N), 0)
    col_ids = lax.broadcasted_iota(jnp.int32, (TILE_A, N), 1)
    d2 = jnp.where(row_ids == col_ids, jnp.inf, d2)

    bits = lax.bitcast_convert_type(d2, jnp.int32)

    bits16 = lax.shift_right_logical(bits, 16).astype(jnp.int16)

    def bisect16_body(_, state):
        lo, hi = state
        mid = lo + lax.shift_right_arithmetic(hi - lo, 1)
        mid16 = mid.astype(jnp.int16)
        cnt = jnp.sum((bits16 <= mid16).astype(jnp.int16), axis=1,
                      keepdims=True).astype(jnp.int32)
        ge = cnt >= K
        return jnp.where(ge, lo, mid + 1), jnp.where(ge, mid, hi)

    lo16 = jnp.full((TILE_A, 1), -1, dtype=jnp.int32)
    hi16 = jnp.full((TILE_A, 1), 0x7F7F, dtype=jnp.int32)
    _, v16i = lax.fori_loop(0, 15, bisect16_body, (lo16, hi16))

    def bisect_body(_, state):
        lo, hi = state
        mid = lo + lax.shift_right_logical(hi - lo, 1)
        cnt = jnp.sum((bits <= mid).astype(jnp.int32), axis=1, keepdims=True)
        ge = cnt >= K
        return jnp.where(ge, lo, mid + 1), jnp.where(ge, mid, hi)

    lo0 = lax.shift_left(v16i, 16) - 1
    hi0 = lax.shift_left(v16i + 1, 16) - 1
    _, vbits = lax.fori_loop(0, 16, bisect_body, (lo0, hi0))

    maskf = jnp.where(bits <= vbits, 1.0, 0.0)

    i128a = lax.broadcasted_iota(jnp.int32, (NB, NB), 0)
    i128b = lax.broadcasted_iota(jnp.int32, (NB, NB), 1)
    T = jnp.where(i128a < i128b, 1.0, 0.0)
    m2 = maskf.reshape(TILE_A * NBLK, NB)
    p_in = lax.dot_general(
        m2, T, (((1,), (0,)), ((), ())),
        preferred_element_type=jnp.float32).reshape(TILE_A, N)

    colb = lax.shift_right_logical(
        lax.broadcasted_iota(jnp.int32, (N, NBLK), 0), 7)
    blkid = lax.broadcasted_iota(jnp.int32, (N, NBLK), 1)
    C = jnp.where(colb == blkid, 1.0, 0.0)
    t32 = lax.dot_general(
        maskf, C, (((1,), (0,)), ((), ())),
        preferred_element_type=jnp.float32)
    ia = lax.broadcasted_iota(jnp.int32, (NBLK, NBLK), 0)
    ib = lax.broadcasted_iota(jnp.int32, (NBLK, NBLK), 1)
    S = jnp.where(ia < ib, 1.0, 0.0)
    base32 = lax.dot_general(
        t32, S, (((1,), (0,)), ((), ())),
        preferred_element_type=jnp.float32)
    rowb = lax.broadcasted_iota(jnp.int32, (NBLK, N), 0)
    colb2 = lax.shift_right_logical(
        lax.broadcasted_iota(jnp.int32, (NBLK, N), 1), 7)
    E = jnp.where(rowb == colb2, 1.0, 0.0)
    base = lax.dot_general(
        base32, E, (((1,), (0,)), ((), ())),
        preferred_element_type=jnp.float32)

    rank = base + p_in
    sel = jnp.logical_and(maskf > 0.0, rank < float(PAD))
    rankv = jnp.where(sel, rank, -1.0)

    labs = bl_ref[...]
    packed = lax.bitcast_convert_type(
        jnp.bitwise_or(jnp.bitwise_and(bits, ~7), labs), jnp.float32)

    out_ref[...] = jnp.concatenate([packed, rankv], axis=1)


def _tc_a(X, XT, BL, g):
    gblk = GN // TILE_A
    return pl.pallas_call(
        functools.partial(_tc_a_kernel, g * GN),
        grid=(gblk,),
        in_specs=[
            pl.BlockSpec((TILE_A, D), lambda i, _g=g: (i + _g * gblk, 0)),
            pl.BlockSpec((D, N), lambda i: (0, 0)),
            pl.BlockSpec((1, N), lambda i: (0, 0)),
        ],
        out_specs=pl.BlockSpec((TILE_A, XW), lambda i: (i, 0)),
        out_shape=jax.ShapeDtypeStruct((GN, XW), jnp.float32),
    )(X, XT, BL)


def _sc_body(d2x_hbm, out_hbm, rowbuf0, rowbuf1, outbuf, sem0, sem1):
    wid = lax.axis_index("s") * 2 + lax.axis_index("c")
    base = wid * RPW

    def prefill(r, _):
        def inner(j, _):
            outbuf[r, pl.ds(j * L, L)] = jnp.full((L,), _PADVAL, jnp.float32)
            return 0
        return lax.fori_loop(0, PAD // L, inner, 0)
    lax.fori_loop(0, RPW, prefill, 0)

    def process(buf, lr):
        rsplat = jnp.broadcast_to(lr, (L,)).astype(jnp.int32)

        @plsc.parallel_loop(0, N // L, unroll=8)
        def chunk(c):
            vals = buf[pl.ds(c * L, L)]
            rk = buf[pl.ds(N + c * L, L)]
            m = rk >= 0.0
            idx = jnp.maximum(rk, 0.0).astype(jnp.int32)
            plsc.store_scatter(outbuf, [rsplat, idx], vals, mask=m)

    pltpu.async_copy(d2x_hbm.at[base], rowbuf0, sem0)

    def pair(p, _):
        r0 = 2 * p
        r1 = r0 + 1
        pltpu.make_async_copy(d2x_hbm.at[base + r0], rowbuf0, sem0).wait()
        pltpu.async_copy(d2x_hbm.at[base + r1], rowbuf1, sem1)
        process(rowbuf0, r0)
        pltpu.make_async_copy(d2x_hbm.at[base + r1], rowbuf1, sem1).wait()

        @pl.when(p < RPW // 2 - 1)
        def _():
            pltpu.async_copy(d2x_hbm.at[base + r0 + 2], rowbuf0, sem0)

        process(rowbuf1, r1)
        return 0

    lax.fori_loop(0, RPW // 2, pair, 0)

    pltpu.sync_copy(outbuf, out_hbm.at[pl.ds(base, RPW)])


def _sc_compact(d2x):
    run = pl.kernel(
        _sc_body,
        out_type=jax.ShapeDtypeStruct((GN, PAD), jnp.float32),
        mesh=plsc.VectorSubcoreMesh(core_axis_name="c", subcore_axis_name="s"),
        scratch_types=(
            pltpu.VMEM((XW,), jnp.float32),
            pltpu.VMEM((XW,), jnp.float32),
            pltpu.VMEM((RPW, PAD), jnp.float32),
            pltpu.SemaphoreType.DMA,
            pltpu.SemaphoreType.DMA,
        ),
        compiler_params=pltpu.CompilerParams(needs_layout_passes=False),
    )
    return run(d2x)


def _tc_c_kernel(d_ref, out_ref):
    pbits = lax.bitcast_convert_type(d_ref[...], jnp.int32)
    labs = jnp.bitwise_and(pbits, 7)
    d2c = lax.bitcast_convert_type(jnp.bitwise_and(pbits, ~7), jnp.float32)
    dist = jnp.sqrt(jnp.maximum(d2c, 1e-12))
    logU = jnp.log(jnp.float32(PERPLEXITY))

    def hbeta(beta):
        E = jnp.exp(-dist * beta)
        Ps = jnp.sum(E, axis=1, keepdims=True)
        S = jnp.sum(dist * E, axis=1, keepdims=True)
        Ps_safe = jnp.where(Ps > 0, Ps, 1.0)
        H = jnp.where(Ps > 0, jnp.log(Ps_safe) + beta * S / Ps_safe, 0.0)
        return E, Ps, Ps_safe, H

    def search_body(_, state):
        beta, bmin, bmax = state
        _, _, _, H = hbeta(beta)
        Hdiff = H - logU
        done = jnp.abs(Hdiff) < TOL
        bmin_n = jnp.where(Hdiff > 0, beta, bmin)
        bmax_n = jnp.where(Hdiff > 0, bmax, beta)
        beta_up = jnp.where(jnp.isfinite(bmax), (beta + bmax) / 2.0, beta * 2.0)
        beta_dn = jnp.where(jnp.isfinite(bmin), (beta + bmin) / 2.0, beta / 2.0)
        beta_n = jnp.where(Hdiff > 0, beta_up, beta_dn)
        beta_n = jnp.where(done, beta, beta_n)
        bmin_n = jnp.where(done, bmin, bmin_n)
        bmax_n = jnp.where(done, bmax, bmax_n)
        return beta_n, bmin_n, bmax_n

    init = (jnp.full((TILE_C, 1), 1.0, jnp.float32),
            jnp.full((TILE_C, 1), -jnp.inf, jnp.float32),
            jnp.full((TILE_C, 1), jnp.inf, jnp.float32))
    beta, _, _ = lax.fori_loop(0, N_TRIES, search_body, init)

    E, Ps, Ps_safe, H = hbeta(beta)
    P = jnp.where(Ps > 0, E / Ps_safe, 0.0)

    simpson = jnp.zeros((TILE_C, 1), jnp.float32)
    for c in range(N_CATEGORIES):
        pc = jnp.sum(jnp.where(labs == c, P, 0.0), axis=1, keepdims=True)
        simpson = simpson + pc * pc
    simpson = jnp.where(H == 0.0, -1.0, simpson)
    out_ref[...] = 1.0 / simpson


def _tc_c(dcomp):
    return pl.pallas_call(
        _tc_c_kernel,
        grid=(GN // TILE_C,),
        in_specs=[pl.BlockSpec((TILE_C, PAD), lambda i: (i, 0))],
        out_specs=pl.BlockSpec((TILE_C, 1), lambda i: (i, 0)),
        out_shape=jax.ShapeDtypeStruct((GN, 1), jnp.float32),
    )(dcomp)


@jax.jit
def kernel(X, batch_id):
    XT = X.T
    BL = batch_id.astype(jnp.int32).reshape(1, N)
    parts = []
    for g in range(G):
        d2x = _tc_a(X, XT, BL, g)
        dcomp = _sc_compact(d2x)
        parts.append(_tc_c(dcomp))
    return jnp.concatenate(parts, axis=0).reshape(N)


# speedup vs baseline: 1.2153x; 1.2153x over previous
import functools

import jax
import jax.numpy as jnp
from jax import lax
from jax.experimental import pallas as pl
from jax.experimental.pallas import tpu as pltpu, tpu_sc as plsc

N = 4096
D = 64
PERPLEXITY = 30.0
TOL = 1e-05
K = int(PERPLEXITY * 3)
N_CATEGORIES = 8
N_TRIES = 50

G = 4
GN = N // G
TILE_A = 256
TILE_C = 512
PAD = 128
XW = 2 * N
NW = 32
RPW = GN // NW
L = 16
NB = 128
NBLK = N // NB

_MAX_FINITE_BITS = 0x7F7FFFFF
_PADVAL = 1e38


def _tc_a_kernel(goff, x_ref, xt_ref, bl_ref, out_ref):
    pid = pl.program_id(0)
    x = x_ref[...]
    xt = xt_ref[...]

    dot = lax.dot_general(
        x, xt, (((1,), (0,)), ((), ())), preferred_element_type=jnp.float32)
    sq_row = jnp.sum(x * x, axis=1, keepdims=True)
    sq_lane = jnp.sum(xt * xt, axis=0, keepdims=True)
    d2 = jnp.maximum(sq_row + sq_lane - 2.0 * dot, 0.0)

    row_ids = goff + pid * TILE_A + lax.broadcasted_iota(
        jnp.int32, (TILE_A, N), 0)
    col_ids = lax.broadcasted_iota(jnp.int32, (TILE_A, N), 1)
    d2 = jnp.where(row_ids == col_ids, jnp.inf, d2)

    bits = lax.bitcast_convert_type(d2, jnp.int32)

    def bisect_body(_, state):
        lo, hi = state
        mid = lo + lax.shift_right_logical(hi - lo, 1)
        cnt = jnp.sum((bits <= mid).astype(jnp.int32), axis=1, keepdims=True)
        ge = cnt >= K
        return jnp.where(ge, lo, mid + 1), jnp.where(ge, mid, hi)

    lo0 = jnp.full((TILE_A, 1), -1, dtype=jnp.int32)
    hi0 = jnp.full((TILE_A, 1), _MAX_FINITE_BITS, dtype=jnp.int32)
    _, vbits = lax.fori_loop(0, 31, bisect_body, (lo0, hi0))

    maskf = jnp.where(bits <= vbits, 1.0, 0.0)

    i128a = lax.broadcasted_iota(jnp.int32, (NB, NB), 0)
    i128b = lax.broadcasted_iota(jnp.int32, (NB, NB), 1)
    T = jnp.where(i128a < i128b, 1.0, 0.0)
    m2 = maskf.reshape(TILE_A * NBLK, NB)
    p_in = lax.dot_general(
        m2, T, (((1,), (0,)), ((), ())),
        preferred_element_type=jnp.float32).reshape(TILE_A, N)

    colb = lax.shift_right_logical(
        lax.broadcasted_iota(jnp.int32, (N, NBLK), 0), 7)
    blkid = lax.broadcasted_iota(jnp.int32, (N, NBLK), 1)
    C = jnp.where(colb == blkid, 1.0, 0.0)
    t32 = lax.dot_general(
        maskf, C, (((1,), (0,)), ((), ())),
        preferred_element_type=jnp.float32)
    ia = lax.broadcasted_iota(jnp.int32, (NBLK, NBLK), 0)
    ib = lax.broadcasted_iota(jnp.int32, (NBLK, NBLK), 1)
    S = jnp.where(ia < ib, 1.0, 0.0)
    base32 = lax.dot_general(
        t32, S, (((1,), (0,)), ((), ())),
        preferred_element_type=jnp.float32)
    rowb = lax.broadcasted_iota(jnp.int32, (NBLK, N), 0)
    colb2 = lax.shift_right_logical(
        lax.broadcasted_iota(jnp.int32, (NBLK, N), 1), 7)
    E = jnp.where(rowb == colb2, 1.0, 0.0)
    base = lax.dot_general(
        base32, E, (((1,), (0,)), ((), ())),
        preferred_element_type=jnp.float32)

    rank = base + p_in
    sel = jnp.logical_and(maskf > 0.0, rank < float(PAD))
    rankv = jnp.where(sel, rank, -1.0)

    labs = bl_ref[...]
    packed = lax.bitcast_convert_type(
        jnp.bitwise_or(jnp.bitwise_and(bits, ~7), labs), jnp.float32)

    out_ref[...] = jnp.concatenate([packed, rankv], axis=1)


def _tc_a(X, XT, BL, g):
    gblk = GN // TILE_A
    return pl.pallas_call(
        functools.partial(_tc_a_kernel, g * GN),
        grid=(gblk,),
        in_specs=[
            pl.BlockSpec((TILE_A, D), lambda i, _g=g: (i + _g * gblk, 0)),
            pl.BlockSpec((D, N), lambda i: (0, 0)),
            pl.BlockSpec((1, N), lambda i: (0, 0)),
        ],
        out_specs=pl.BlockSpec((TILE_A, XW), lambda i: (i, 0)),
        out_shape=jax.ShapeDtypeStruct((GN, XW), jnp.float32),
    )(X, XT, BL)


def _sc_body(d2x_hbm, out_hbm, rowbuf0, rowbuf1, outbuf, sem0, sem1):
    wid = lax.axis_index("s") * 2 + lax.axis_index("c")
    base = wid * RPW

    def prefill(r, _):
        def inner(j, _):
            outbuf[r, pl.ds(j * L, L)] = jnp.full((L,), _PADVAL, jnp.float32)
            return 0
        return lax.fori_loop(0, PAD // L, inner, 0)
    lax.fori_loop(0, RPW, prefill, 0)

    def process(buf, lr):
        rsplat = jnp.broadcast_to(lr, (L,)).astype(jnp.int32)

        @plsc.parallel_loop(0, N // L, unroll=8)
        def chunk(c):
            vals = buf[pl.ds(c * L, L)]
            rk = buf[pl.ds(N + c * L, L)]
            m = rk >= 0.0
            idx = jnp.maximum(rk, 0.0).astype(jnp.int32)
            plsc.store_scatter(outbuf, [rsplat, idx], vals, mask=m)

    pltpu.async_copy(d2x_hbm.at[base], rowbuf0, sem0)

    def pair(p, _):
        r0 = 2 * p
        r1 = r0 + 1
        pltpu.make_async_copy(d2x_hbm.at[base + r0], rowbuf0, sem0).wait()
        pltpu.async_copy(d2x_hbm.at[base + r1], rowbuf1, sem1)
        process(rowbuf0, r0)
        pltpu.make_async_copy(d2x_hbm.at[base + r1], rowbuf1, sem1).wait()

        @pl.when(p < RPW // 2 - 1)
        def _():
            pltpu.async_copy(d2x_hbm.at[base + r0 + 2], rowbuf0, sem0)

        process(rowbuf1, r1)
        return 0

    lax.fori_loop(0, RPW // 2, pair, 0)

    pltpu.sync_copy(outbuf, out_hbm.at[pl.ds(base, RPW)])


def _sc_compact(d2x):
    run = pl.kernel(
        _sc_body,
        out_type=jax.ShapeDtypeStruct((GN, PAD), jnp.float32),
        mesh=plsc.VectorSubcoreMesh(core_axis_name="c", subcore_axis_name="s"),
        scratch_types=(
            pltpu.VMEM((XW,), jnp.float32),
            pltpu.VMEM((XW,), jnp.float32),
            pltpu.VMEM((RPW, PAD), jnp.float32),
            pltpu.SemaphoreType.DMA,
            pltpu.SemaphoreType.DMA,
        ),
        compiler_params=pltpu.CompilerParams(needs_layout_passes=False),
    )
    return run(d2x)


def _tc_c_kernel(d_ref, out_ref):
    pbits = lax.bitcast_convert_type(d_ref[...], jnp.int32)
    labs = jnp.bitwise_and(pbits, 7)
    d2c = lax.bitcast_convert_type(jnp.bitwise_and(pbits, ~7), jnp.float32)
    dist = jnp.sqrt(jnp.maximum(d2c, 1e-12))
    logU = jnp.log(jnp.float32(PERPLEXITY))

    def hbeta(beta):
        E = jnp.exp(-dist * beta)
        Ps = jnp.sum(E, axis=1, keepdims=True)
        S = jnp.sum(dist * E, axis=1, keepdims=True)
        Ps_safe = jnp.where(Ps > 0, Ps, 1.0)
        H = jnp.where(Ps > 0, jnp.log(Ps_safe) + beta * S / Ps_safe, 0.0)
        return E, Ps, Ps_safe, H

    def search_body(_, state):
        beta, bmin, bmax = state
        _, _, _, H = hbeta(beta)
        Hdiff = H - logU
        done = jnp.abs(Hdiff) < TOL
        bmin_n = jnp.where(Hdiff > 0, beta, bmin)
        bmax_n = jnp.where(Hdiff > 0, bmax, beta)
        beta_up = jnp.where(jnp.isfinite(bmax), (beta + bmax) / 2.0, beta * 2.0)
        beta_dn = jnp.where(jnp.isfinite(bmin), (beta + bmin) / 2.0, beta / 2.0)
        beta_n = jnp.where(Hdiff > 0, beta_up, beta_dn)
        beta_n = jnp.where(done, beta, beta_n)
        bmin_n = jnp.where(done, bmin, bmin_n)
        bmax_n = jnp.where(done, bmax, bmax_n)
        return beta_n, bmin_n, bmax_n

    init = (jnp.full((TILE_C, 1), 1.0, jnp.float32),
            jnp.full((TILE_C, 1), -jnp.inf, jnp.float32),
            jnp.full((TILE_C, 1), jnp.inf, jnp.float32))
    beta, _, _ = lax.fori_loop(0, N_TRIES, search_body, init)

    E, Ps, Ps_safe, H = hbeta(beta)
    P = jnp.where(Ps > 0, E / Ps_safe, 0.0)

    simpson = jnp.zeros((TILE_C, 1), jnp.float32)
    for c in range(N_CATEGORIES):
        pc = jnp.sum(jnp.where(labs == c, P, 0.0), axis=1, keepdims=True)
        simpson = simpson + pc * pc
    simpson = jnp.where(H == 0.0, -1.0, simpson)
    out_ref[...] = 1.0 / simpson


def _tc_c(dcomp):
    return pl.pallas_call(
        _tc_c_kernel,
        grid=(GN // TILE_C,),
        in_specs=[pl.BlockSpec((TILE_C, PAD), lambda i: (i, 0))],
        out_specs=pl.BlockSpec((TILE_C, 1), lambda i: (i, 0)),
        out_shape=jax.ShapeDtypeStruct((GN, 1), jnp.float32),
    )(dcomp)


@jax.jit
def kernel(X, batch_id):
    XT = X.T
    BL = batch_id.astype(jnp.int32).reshape(1, N)
    parts = []
    for g in range(G):
        d2x = _tc_a(X, XT, BL, g)
        dcomp = _sc_compact(d2x)
        parts.append(_tc_c(dcomp))
    return jnp.concatenate(parts, axis=0).reshape(N)
